# baseline (speedup 1.0000x reference)
import functools

import jax
import jax.numpy as jnp
from jax import lax
from jax.experimental import pallas as pl
from jax.experimental.pallas import tpu as pltpu
from jax.experimental.pallas import tpu_sc as plsc

N = 10000
E = 320000
D = 128
NCLS = 64
NC = 2
NS = 16
NW = NC * NS
CHUNK = 64
EPW = 10240
E_PAD = EPW * NW
NCHUNK = EPW // CHUNK
NHALF = 4
HCHUNK = NCHUNK // NHALF
NPAD = 10112
RPT = NPAD // NS
SLC = CHUNK
RPT_FULL = RPT // SLC
RPT_REM = RPT - RPT_FULL * SLC
ROW_BLK = 2000

_mesh = plsc.VectorSubcoreMesh(core_axis_name="c", subcore_axis_name="s")


@functools.partial(
    pl.kernel,
    out_type=jax.ShapeDtypeStruct((NC, NPAD, D), jnp.float32),
    mesh=_mesh,
    scratch_types=[
        pltpu.VMEM((HCHUNK, CHUNK), jnp.int32),
        pltpu.VMEM((HCHUNK, CHUNK), jnp.int32),
        pltpu.VMEM((CHUNK, D), jnp.float32),
        pltpu.VMEM((CHUNK, D), jnp.float32),
        pltpu.VMEM((CHUNK, D), jnp.float32),
        pltpu.VMEM((CHUNK, D), jnp.float32),
        pltpu.VMEM_SHARED((NPAD, D), jnp.float32),
        pltpu.SemaphoreType.DMA,
        pltpu.SemaphoreType.DMA,
        pltpu.SemaphoreType.DMA,
        pltpu.SemaphoreType.DMA,
    ],
)
def _prop_kernel(y_hbm, src_hbm, dst_hbm, out_hbm, src_v, dst_v,
                 buf0, buf1, buf2, buf3, acc, sem0, sem1, sem2, sem3):
    c = lax.axis_index("c")
    s = lax.axis_index("s")
    wid = s * NC + c
    bufs = (buf0, buf1, buf2, buf3)
    sems = (sem0, sem1, sem2, sem3)
    zero16 = jnp.zeros((16,), jnp.float32)

    def zrow(i, _):
        for j in range(D // 16):
            buf0[i, pl.ds(j * 16, 16)] = zero16
        return 0

    lax.fori_loop(0, CHUNK, zrow, 0)
    for k in range(RPT_FULL):
        pltpu.sync_copy(buf0, acc.at[pl.ds(s * RPT + k * SLC, SLC)])
    pltpu.sync_copy(buf0.at[pl.ds(0, RPT_REM)],
                    acc.at[pl.ds(s * RPT + RPT_FULL * SLC, RPT_REM)])
    plsc.subcore_barrier()

    def gather(j, t):
        jn = jnp.minimum(j, HCHUNK - 1)
        pltpu.async_copy(y_hbm.at[src_v.at[jn]], bufs[t], sems[t])

    def gwait(t):
        pltpu.make_async_copy(y_hbm.at[src_v.at[0]], bufs[t], sems[t]).wait()

    def scat(j, t):
        pltpu.sync_copy(bufs[t], acc.at[dst_v.at[j]], add=True)

    def body(jj, _):
        j = jj * 4
        gather(j + 3, 3)
        gwait(0)
        scat(j, 0)
        gather(j + 4, 0)
        gwait(1)
        scat(j + 1, 1)
        gather(j + 5, 1)
        gwait(2)
        scat(j + 2, 2)
        gather(j + 6, 2)
        gwait(3)
        scat(j + 3, 3)
        return 0

    for half in range(NHALF):
        pltpu.sync_copy(src_hbm.at[wid, half], src_v)
        pltpu.sync_copy(dst_hbm.at[wid, half], dst_v)
        gather(0, 0)
        gather(1, 1)
        gather(2, 2)
        lax.fori_loop(0, HCHUNK // 4 - 1, body, 0)
        gather(HCHUNK - 1, 3)
        gwait(0)
        scat(HCHUNK - 4, 0)
        gwait(1)
        scat(HCHUNK - 3, 1)
        gwait(2)
        scat(HCHUNK - 2, 2)
        gwait(3)
        scat(HCHUNK - 1, 3)
    plsc.subcore_barrier()
    for k in range(RPT_FULL):
        r = s * RPT + k * SLC
        pltpu.sync_copy(acc.at[pl.ds(r, SLC)], out_hbm.at[c, pl.ds(r, SLC)])
    r = s * RPT + RPT_FULL * SLC
    pltpu.sync_copy(acc.at[pl.ds(r, RPT_REM)], out_hbm.at[c, pl.ds(r, RPT_REM)])


@functools.partial(
    pl.kernel,
    out_type=jax.ShapeDtypeStruct((NC, NPAD, D), jnp.float32),
    mesh=_mesh,
    scratch_types=[
        pltpu.VMEM((HCHUNK, CHUNK), jnp.int32),
        pltpu.VMEM((CHUNK, D), jnp.float32),
        pltpu.VMEM_SHARED((NPAD, D), jnp.float32),
    ],
)
def _deg_kernel(dst_hbm, out_hbm, dst_v, ones_b, dacc):
    c = lax.axis_index("c")
    s = lax.axis_index("s")
    wid = s * NC + c
    zero16 = jnp.zeros((16,), jnp.float32)
    one16 = jnp.full((16,), 1.0, jnp.float32)

    def zrow(i, _):
        for j in range(D // 16):
            ones_b[i, pl.ds(j * 16, 16)] = zero16
        return 0

    lax.fori_loop(0, CHUNK, zrow, 0)
    for k in range(RPT_FULL):
        pltpu.sync_copy(ones_b, dacc.at[pl.ds(s * RPT + k * SLC, SLC)])
    pltpu.sync_copy(ones_b.at[pl.ds(0, RPT_REM)],
                    dacc.at[pl.ds(s * RPT + RPT_FULL * SLC, RPT_REM)])

    def orow(i, _):
        for j in range(D // 16):
            ones_b[i, pl.ds(j * 16, 16)] = one16
        return 0

    lax.fori_loop(0, CHUNK, orow, 0)
    plsc.subcore_barrier()

    def body(j, _):
        pltpu.sync_copy(ones_b, dacc.at[dst_v.at[j]], add=True)
        return 0

    for half in range(NHALF):
        pltpu.sync_copy(dst_hbm.at[wid, half], dst_v)
        lax.fori_loop(0, HCHUNK, body, 0)
    plsc.subcore_barrier()
    for k in range(RPT_FULL):
        r = s * RPT + k * SLC
        pltpu.sync_copy(dacc.at[pl.ds(r, SLC)], out_hbm.at[c, pl.ds(r, SLC)])
    r = s * RPT + RPT_FULL * SLC
    pltpu.sync_copy(dacc.at[pl.ds(r, RPT_REM)], out_hbm.at[c, pl.ds(r, RPT_REM)])


def _dinv_of(degp_ref):
    deg = degp_ref[0, :, 0:1] + degp_ref[1, :, 0:1] + 1.0
    return lax.rsqrt(deg)


def _k1_body(x_ref, w_ref, degp_ref, y_ref):
    dinv = _dinv_of(degp_ref)
    y_ref[...] = dinv * jnp.dot(x_ref[...], w_ref[...],
                                preferred_element_type=jnp.float32)


def _k2_body(a_ref, y_ref, res_ref, b_ref, degp_ref, w_ref, h_ref, y2_ref):
    dinv = _dinv_of(degp_ref)
    tot = a_ref[0] + a_ref[1] + y_ref[...]
    h = jnp.maximum(dinv * tot + b_ref[...], 0.0) + res_ref[...]
    h_ref[...] = h
    y2_ref[...] = dinv * jnp.dot(h, w_ref[...],
                                 preferred_element_type=jnp.float32)


def _k3_body(a_ref, y_ref, res_ref, b_ref, degp_ref, w_ref, blin_ref, o_ref):
    dinv = _dinv_of(degp_ref)
    tot = a_ref[0] + a_ref[1] + y_ref[...]
    h = jnp.maximum(dinv * tot + b_ref[...], 0.0) + res_ref[...]
    o_ref[...] = jnp.dot(h, w_ref[...],
                         preferred_element_type=jnp.float32) + blin_ref[...]


_GRID = (N // ROW_BLK,)
_row_spec = pl.BlockSpec((ROW_BLK, D), lambda i: (i, 0))
_degp_spec = pl.BlockSpec((NC, ROW_BLK, D), lambda i: (0, i, 0))
_acc_spec = pl.BlockSpec((NC, ROW_BLK, D), lambda i: (0, i, 0))
_w_spec = pl.BlockSpec((D, D), lambda i: (0, 0))
_b_spec = pl.BlockSpec((1, D), lambda i: (0, 0))

_k1_call = pl.pallas_call(
    _k1_body,
    grid=_GRID,
    in_specs=[_row_spec, _w_spec, _degp_spec],
    out_specs=_row_spec,
    out_shape=jax.ShapeDtypeStruct((N, D), jnp.float32),
)

_k2_call = pl.pallas_call(
    _k2_body,
    grid=_GRID,
    in_specs=[_acc_spec, _row_spec, _row_spec, _b_spec, _degp_spec, _w_spec],
    out_specs=[_row_spec, _row_spec],
    out_shape=[jax.ShapeDtypeStruct((N, D), jnp.float32),
               jax.ShapeDtypeStruct((N, D), jnp.float32)],
)

_k3_call = pl.pallas_call(
    _k3_body,
    grid=_GRID,
    in_specs=[_acc_spec, _row_spec, _row_spec, _b_spec, _degp_spec,
              pl.BlockSpec((D, NCLS), lambda i: (0, 0)),
              pl.BlockSpec((1, NCLS), lambda i: (0, 0))],
    out_specs=pl.BlockSpec((ROW_BLK, NCLS), lambda i: (i, 0)),
    out_shape=jax.ShapeDtypeStruct((N, NCLS), jnp.float32),
)


def kernel(x, edge_index, W1, b1, W2, b2, Wlin, blin):
    src = edge_index[0].astype(jnp.int32)
    dst = edge_index[1].astype(jnp.int32)
    pad = E_PAD - E
    srcp = jnp.concatenate([src, jnp.zeros((pad,), jnp.int32)])
    srcp = srcp.reshape(NW, NHALF, HCHUNK, CHUNK)
    trash = N + (jnp.arange(pad, dtype=jnp.int32) % (NPAD - N))
    dstp = jnp.concatenate([dst, trash])
    dstp = dstp.reshape(NW, NHALF, HCHUNK, CHUNK)

    degp = _deg_kernel(dstp)
    y1 = _k1_call(x, W1, degp)
    a1 = _prop_kernel(y1, srcp, dstp)
    h1, y2 = _k2_call(a1, y1, x, b1.reshape(1, D), degp, W2)
    a2 = _prop_kernel(y2, srcp, dstp)
    out = _k3_call(a2, y2, h1, b2.reshape(1, D), degp, Wlin,
                   blin.reshape(1, NCLS))
    return out


# speedup vs baseline: 1.3610x; 1.3610x over previous
import functools

import jax
import jax.numpy as jnp
from jax import lax
from jax.experimental import pallas as pl
from jax.experimental.pallas import tpu as pltpu
from jax.experimental.pallas import tpu_sc as plsc

N = 10000
E = 320000
D = 128
NCLS = 64
NC = 2
NS = 16
NW = NC * NS
CHUNK = 64
EPW = 10240
E_PAD = EPW * NW
NCHUNK = EPW // CHUNK
NHALF = 2
HCHUNK = NCHUNK // NHALF
NPAD = 10112
RPT = NPAD // NS
SLC = CHUNK
RPT_FULL = RPT // SLC
RPT_REM = RPT - RPT_FULL * SLC
ROW_BLK = 2000
CDH = D // 2

_mesh = plsc.VectorSubcoreMesh(core_axis_name="c", subcore_axis_name="s")


@functools.partial(
    pl.kernel,
    out_type=jax.ShapeDtypeStruct((NC, NPAD, D), jnp.float32),
    mesh=_mesh,
    scratch_types=[
        pltpu.VMEM((HCHUNK, CHUNK), jnp.int32),
        pltpu.VMEM((HCHUNK, CHUNK), jnp.int32),
        pltpu.VMEM((CHUNK, CDH), jnp.uint32),
        pltpu.VMEM((CHUNK, CDH), jnp.uint32),
        pltpu.VMEM((CHUNK, CDH), jnp.uint32),
        pltpu.VMEM((CHUNK, CDH), jnp.uint32),
        pltpu.VMEM((CHUNK, D), jnp.float32),
        pltpu.VMEM_SHARED((NPAD, D), jnp.float32),
        pltpu.SemaphoreType.DMA,
        pltpu.SemaphoreType.DMA,
        pltpu.SemaphoreType.DMA,
        pltpu.SemaphoreType.DMA,
    ],
    compiler_params=pltpu.CompilerParams(use_tc_tiling_on_sc=False),
)
def _prop_kernel(y_hbm, src_hbm, dst_hbm, out_hbm, src_v, dst_v,
                 buf0, buf1, buf2, buf3, fbuf, acc, sem0, sem1, sem2, sem3):
    c = lax.axis_index("c")
    s = lax.axis_index("s")
    wid = s * NC + c
    bufs = (buf0, buf1, buf2, buf3)
    sems = (sem0, sem1, sem2, sem3)
    zero16 = jnp.zeros((16,), jnp.float32)

    def zrow(i, _):
        for j in range(D // 16):
            fbuf[i, pl.ds(j * 16, 16)] = zero16
        return 0

    lax.fori_loop(0, CHUNK, zrow, 0)
    for k in range(RPT_FULL):
        pltpu.sync_copy(fbuf, acc.at[pl.ds(s * RPT + k * SLC, SLC)])
    pltpu.sync_copy(fbuf.at[pl.ds(0, RPT_REM)],
                    acc.at[pl.ds(s * RPT + RPT_FULL * SLC, RPT_REM)])
    plsc.subcore_barrier()

    def gather(j, t):
        jn = jnp.minimum(j, HCHUNK - 1)
        pltpu.async_copy(y_hbm.at[src_v.at[jn]], bufs[t], sems[t])

    def gwait(t):
        pltpu.make_async_copy(y_hbm.at[src_v.at[0]], bufs[t], sems[t]).wait()

    def conv_scat(j, t):
        def crow(i, _):
            for m in range(CDH // 16):
                x = bufs[t][i, pl.ds(m * 16, 16)]
                fbuf[i, pl.ds(m * 16, 16)] = lax.bitcast_convert_type(
                    x << 16, jnp.float32)
                fbuf[i, pl.ds(CDH + m * 16, 16)] = lax.bitcast_convert_type(
                    x & jnp.uint32(0xFFFF0000), jnp.float32)
            return 0

        lax.fori_loop(0, CHUNK, crow, 0)
        pltpu.sync_copy(fbuf, acc.at[dst_v.at[j]], add=True)

    def body(jj, _):
        j = jj * 4
        gather(j + 3, 3)
        gwait(0)
        conv_scat(j, 0)
        gather(j + 4, 0)
        gwait(1)
        conv_scat(j + 1, 1)
        gather(j + 5, 1)
        gwait(2)
        conv_scat(j + 2, 2)
        gather(j + 6, 2)
        gwait(3)
        conv_scat(j + 3, 3)
        return 0

    for half in range(NHALF):
        pltpu.sync_copy(src_hbm.at[wid, half], src_v)
        pltpu.sync_copy(dst_hbm.at[wid, half], dst_v)
        gather(0, 0)
        gather(1, 1)
        gather(2, 2)
        lax.fori_loop(0, HCHUNK // 4 - 1, body, 0)
        gather(HCHUNK - 1, 3)
        gwait(0)
        conv_scat(HCHUNK - 4, 0)
        gwait(1)
        conv_scat(HCHUNK - 3, 1)
        gwait(2)
        conv_scat(HCHUNK - 2, 2)
        gwait(3)
        conv_scat(HCHUNK - 1, 3)
    plsc.subcore_barrier()
    for k in range(RPT_FULL):
        r = s * RPT + k * SLC
        pltpu.sync_copy(acc.at[pl.ds(r, SLC)], out_hbm.at[c, pl.ds(r, SLC)])
    r = s * RPT + RPT_FULL * SLC
    pltpu.sync_copy(acc.at[pl.ds(r, RPT_REM)], out_hbm.at[c, pl.ds(r, RPT_REM)])


@functools.partial(
    pl.kernel,
    out_type=jax.ShapeDtypeStruct((NC, NPAD, D), jnp.float32),
    mesh=_mesh,
    scratch_types=[
        pltpu.VMEM((HCHUNK, CHUNK), jnp.int32),
        pltpu.VMEM((CHUNK, D), jnp.float32),
        pltpu.VMEM_SHARED((NPAD, D), jnp.float32),
    ],
)
def _deg_kernel(dst_hbm, out_hbm, dst_v, ones_b, dacc):
    c = lax.axis_index("c")
    s = lax.axis_index("s")
    wid = s * NC + c
    zero16 = jnp.zeros((16,), jnp.float32)
    one16 = jnp.full((16,), 1.0, jnp.float32)

    def zrow(i, _):
        for j in range(D // 16):
            ones_b[i, pl.ds(j * 16, 16)] = zero16
        return 0

    lax.fori_loop(0, CHUNK, zrow, 0)
    for k in range(RPT_FULL):
        pltpu.sync_copy(ones_b, dacc.at[pl.ds(s * RPT + k * SLC, SLC)])
    pltpu.sync_copy(ones_b.at[pl.ds(0, RPT_REM)],
                    dacc.at[pl.ds(s * RPT + RPT_FULL * SLC, RPT_REM)])

    def orow(i, _):
        for j in range(D // 16):
            ones_b[i, pl.ds(j * 16, 16)] = one16
        return 0

    lax.fori_loop(0, CHUNK, orow, 0)
    plsc.subcore_barrier()

    def body(j, _):
        pltpu.sync_copy(ones_b, dacc.at[dst_v.at[j]], add=True)
        return 0

    for half in range(NHALF):
        pltpu.sync_copy(dst_hbm.at[wid, half], dst_v)
        lax.fori_loop(0, HCHUNK, body, 0)
    plsc.subcore_barrier()
    for k in range(RPT_FULL):
        r = s * RPT + k * SLC
        pltpu.sync_copy(dacc.at[pl.ds(r, SLC)], out_hbm.at[c, pl.ds(r, SLC)])
    r = s * RPT + RPT_FULL * SLC
    pltpu.sync_copy(dacc.at[pl.ds(r, RPT_REM)], out_hbm.at[c, pl.ds(r, RPT_REM)])


def _dinv_of(degp_ref):
    deg = degp_ref[0, :, 0:1] + degp_ref[1, :, 0:1] + 1.0
    return lax.rsqrt(deg)


def _pack_pairs(y):
    yu = lax.bitcast_convert_type(y, jnp.uint32)
    lo = (yu[:, :CDH] + jnp.uint32(0x8000)) >> 16
    hi = (yu[:, CDH:] + jnp.uint32(0x8000)) & jnp.uint32(0xFFFF0000)
    return hi | lo


def _k1_body(x_ref, w_ref, degp_ref, y_ref, yp_ref):
    dinv = _dinv_of(degp_ref)
    y = dinv * jnp.dot(x_ref[...], w_ref[...],
                       preferred_element_type=jnp.float32)
    y_ref[...] = y
    yp_ref[...] = _pack_pairs(y)


def _k2_body(a_ref, y_ref, res_ref, b_ref, degp_ref, w_ref,
             h_ref, y2_ref, y2p_ref):
    dinv = _dinv_of(degp_ref)
    tot = a_ref[0] + a_ref[1] + y_ref[...]
    h = jnp.maximum(dinv * tot + b_ref[...], 0.0) + res_ref[...]
    h_ref[...] = h
    y2 = dinv * jnp.dot(h, w_ref[...], preferred_element_type=jnp.float32)
    y2_ref[...] = y2
    y2p_ref[...] = _pack_pairs(y2)


def _k3_body(a_ref, y_ref, res_ref, b_ref, degp_ref, w_ref, blin_ref, o_ref):
    dinv = _dinv_of(degp_ref)
    tot = a_ref[0] + a_ref[1] + y_ref[...]
    h = jnp.maximum(dinv * tot + b_ref[...], 0.0) + res_ref[...]
    o_ref[...] = jnp.dot(h, w_ref[...],
                         preferred_element_type=jnp.float32) + blin_ref[...]


_GRID = (N // ROW_BLK,)
_row_spec = pl.BlockSpec((ROW_BLK, D), lambda i: (i, 0))
_degp_spec = pl.BlockSpec((NC, ROW_BLK, D), lambda i: (0, i, 0))
_acc_spec = pl.BlockSpec((NC, ROW_BLK, D), lambda i: (0, i, 0))
_w_spec = pl.BlockSpec((D, D), lambda i: (0, 0))
_b_spec = pl.BlockSpec((1, D), lambda i: (0, 0))

_pack_spec = pl.BlockSpec((ROW_BLK, CDH), lambda i: (i, 0))
_pack_shape = jax.ShapeDtypeStruct((N, CDH), jnp.uint32)

_k1_call = pl.pallas_call(
    _k1_body,
    grid=_GRID,
    in_specs=[_row_spec, _w_spec, _degp_spec],
    out_specs=[_row_spec, _pack_spec],
    out_shape=[jax.ShapeDtypeStruct((N, D), jnp.float32), _pack_shape],
)

_k2_call = pl.pallas_call(
    _k2_body,
    grid=_GRID,
    in_specs=[_acc_spec, _row_spec, _row_spec, _b_spec, _degp_spec, _w_spec],
    out_specs=[_row_spec, _row_spec, _pack_spec],
    out_shape=[jax.ShapeDtypeStruct((N, D), jnp.float32),
               jax.ShapeDtypeStruct((N, D), jnp.float32), _pack_shape],
)

_k3_call = pl.pallas_call(
    _k3_body,
    grid=_GRID,
    in_specs=[_acc_spec, _row_spec, _row_spec, _b_spec, _degp_spec,
              pl.BlockSpec((D, NCLS), lambda i: (0, 0)),
              pl.BlockSpec((1, NCLS), lambda i: (0, 0))],
    out_specs=pl.BlockSpec((ROW_BLK, NCLS), lambda i: (i, 0)),
    out_shape=jax.ShapeDtypeStruct((N, NCLS), jnp.float32),
)


def kernel(x, edge_index, W1, b1, W2, b2, Wlin, blin):
    src = edge_index[0].astype(jnp.int32)
    dst = edge_index[1].astype(jnp.int32)
    pad = E_PAD - E
    srcp = jnp.concatenate([src, jnp.zeros((pad,), jnp.int32)])
    srcp = srcp.reshape(NW, NHALF, HCHUNK, CHUNK)
    trash = N + (jnp.arange(pad, dtype=jnp.int32) % (NPAD - N))
    dstp = jnp.concatenate([dst, trash])
    dstp = dstp.reshape(NW, NHALF, HCHUNK, CHUNK)

    degp = _deg_kernel(dstp)
    y1, y1p = _k1_call(x, W1, degp)
    a1 = _prop_kernel(y1p, srcp, dstp)
    h1, y2, y2p = _k2_call(a1, y1, x, b1.reshape(1, D), degp, W2)
    a2 = _prop_kernel(y2p, srcp, dstp)
    out = _k3_call(a2, y2, h1, b2.reshape(1, D), degp, Wlin,
                   blin.reshape(1, NCLS))
    return out
